# baseline (speedup 1.0000x reference)
import jax
import jax.numpy as jnp
from jax.experimental import pallas as pl
from jax.experimental.pallas import tpu as pltpu

B, S, D = 32, 4096, 768
K = S // 4
BS = 2048
NBLK = B * S // BS
RPB = S // BS

_I32_MIN = -2147483648
_I32_TOPMASK = 2147483647


def _stage1_body(x_ref, wt_ref, b_ref, q_ref):
    y = jax.lax.dot_general(wt_ref[...], x_ref[0],
                            (((1,), (1,)), ((), ())),
                            preferred_element_type=jnp.float32)
    l0 = y[0:1, :] + b_ref[0, 0]
    l1 = y[1:2, :] + b_ref[0, 1]
    score = l1 - l0
    m = jnp.maximum(l0, l1)
    lse = m + jnp.log(jnp.exp(l0 - m) + jnp.exp(l1 - m))
    lp0 = l0 - lse
    lp1 = l1 - lse
    ent = -(jnp.exp(lp0) * lp0 + jnp.exp(lp1) * lp1)
    q_ref[0] = jnp.concatenate(
        [score, lp0, lp1, ent, jnp.zeros((4, BS), jnp.float32)], axis=0)


def _sortable_key(score):
    bits = jax.lax.bitcast_convert_type(score, jnp.int32)
    return jnp.where(bits < 0, bits ^ jnp.int32(_I32_TOPMASK), bits)


def _stage2_body(q_ref, act_ref, lp_ref, ent_ref):
    score = q_ref[:, 0, :].reshape(B, S)
    lp0 = q_ref[:, 1, :].reshape(B, S)
    lp1 = q_ref[:, 2, :].reshape(B, S)
    ent_ref[...] = q_ref[:, 3, :].reshape(B, S)

    key = _sortable_key(score)

    def bit_step(i, t_u):
        cand = t_u | jnp.left_shift(jnp.int32(1), 31 - i)
        scand = cand ^ jnp.int32(_I32_MIN)
        cnt = jnp.sum((key >= scand).astype(jnp.int32), axis=1, keepdims=True)
        return jnp.where(cnt >= K, cand, t_u)

    t_u = jax.lax.fori_loop(0, 32, bit_step, jnp.zeros((B, 1), jnp.int32))
    thr = t_u ^ jnp.int32(_I32_MIN)

    gt = key > thr
    eq = key == thr
    c_gt = jnp.sum(gt.astype(jnp.int32), axis=1, keepdims=True)
    need = K - c_gt

    idx = jax.lax.broadcasted_iota(jnp.int32, (B, S), 1)

    def idx_step(i, t_j):
        cand = t_j | jnp.left_shift(jnp.int32(1), 11 - i)
        cnt = jnp.sum((eq & (idx <= cand)).astype(jnp.int32), axis=1,
                      keepdims=True)
        return jnp.where(cnt <= need, cand, t_j)

    t_j = jax.lax.fori_loop(0, 12, idx_step, jnp.zeros((B, 1), jnp.int32))

    mask = gt | (eq & (idx <= t_j))
    act_ref[...] = mask.astype(jnp.int32)
    lp_ref[...] = jnp.where(mask, lp1, lp0)


@jax.jit
def kernel(x, W, b):
    x3 = x.reshape(NBLK, BS, D)
    q = pl.pallas_call(
        _stage1_body,
        grid=(NBLK,),
        in_specs=[
            pl.BlockSpec((1, BS, D), lambda i: (i, 0, 0)),
            pl.BlockSpec((2, D), lambda i: (0, 0)),
            pl.BlockSpec((1, 2), lambda i: (0, 0)),
        ],
        out_specs=pl.BlockSpec((1, 8, BS), lambda i: (i, 0, 0)),
        out_shape=jax.ShapeDtypeStruct((NBLK, 8, BS), jnp.float32),
        compiler_params=pltpu.CompilerParams(
            dimension_semantics=("arbitrary",)),
    )(x3, W.T, b.reshape(1, 2))

    fdef = jax.ShapeDtypeStruct((B, S), jnp.float32)
    actions, log_probs, ent = pl.pallas_call(
        _stage2_body,
        out_shape=[jax.ShapeDtypeStruct((B, S), jnp.int32), fdef, fdef],
    )(q)

    topk_log_probs = jnp.zeros((B, S), jnp.float32)
    return (actions, topk_log_probs, log_probs, ent)
